# Initial kernel scaffold; baseline (speedup 1.0000x reference)
import jax
import jax.numpy as jnp
from jax.experimental import pallas as pl


def kernel(x, weight):
    raise NotImplementedError("write your pallas kernel here")


# speedup vs baseline: 1.5751x; 1.5751x over previous
import functools

import jax
import jax.numpy as jnp
from jax import lax
from jax.experimental import pallas as pl
from jax.experimental.pallas import tpu as pltpu
from jax.experimental.pallas import tpu_sc as plsc


@functools.cache
def _build(B, V, D):
    info = plsc.get_sparse_core_info()
    NC, NS = info.num_cores, info.num_subcores
    NW = NC * NS
    assert B % NW == 0
    per_w = B // NW
    C = 128
    assert per_w % C == 0
    n_chunks = per_w // C

    mesh = plsc.VectorSubcoreMesh(core_axis_name="c", subcore_axis_name="s")

    @functools.partial(
        pl.kernel,
        mesh=mesh,
        out_type=jax.ShapeDtypeStruct((B, D), jnp.float32),
        scratch_types=[
            pltpu.VMEM((C,), jnp.int32),
            pltpu.VMEM((C, D), jnp.float32),
            pltpu.SemaphoreType.DMA,
        ],
        compiler_params=pltpu.CompilerParams(use_tc_tiling_on_sc=False),
    )
    def gather_kernel(idx_hbm, table_hbm, out_hbm, idx_v, rows_v, sem):
        wid = lax.axis_index("s") * NC + lax.axis_index("c")
        base = wid * per_w

        def body(g, carry):
            start = base + g * C
            pltpu.sync_copy(idx_hbm.at[pl.ds(start, C)], idx_v)
            pltpu.async_copy(table_hbm.at[idx_v], rows_v, sem).wait()
            pltpu.sync_copy(rows_v, out_hbm.at[pl.ds(start, C)])
            return carry

        lax.fori_loop(0, n_chunks, body, 0)

    return gather_kernel


def kernel(x, weight):
    B = x.shape[0] * x.shape[1]
    V, D = weight.shape
    flat = x.reshape(B)
    out = _build(B, V, D)(flat, weight)
    return out.reshape(x.shape + (D,))


# speedup vs baseline: 1.8772x; 1.1918x over previous
import functools

import jax
import jax.numpy as jnp
from jax import lax
from jax.experimental import pallas as pl
from jax.experimental.pallas import tpu as pltpu
from jax.experimental.pallas import tpu_sc as plsc

_C = 128
_SUP = 4
_NBUF = 2


@functools.cache
def _build(B, V, D):
    info = plsc.get_sparse_core_info()
    NC, NS = info.num_cores, info.num_subcores
    NW = NC * NS
    CH = _SUP * _C
    assert B % (NW * CH) == 0
    per_w = B // NW
    n_chunks = per_w // _C
    n_super = per_w // CH
    assert n_super % _NBUF == 0

    mesh = plsc.VectorSubcoreMesh(core_axis_name="c", subcore_axis_name="s")

    @functools.partial(
        pl.kernel,
        mesh=mesh,
        out_type=jax.ShapeDtypeStruct((B, D), jnp.float32),
        scratch_types=[
            pltpu.VMEM((n_chunks, _C), jnp.int32),
            pltpu.VMEM((_NBUF, CH, D), jnp.float32),
            pltpu.SemaphoreType.DMA((_NBUF,)),
            pltpu.SemaphoreType.DMA((_NBUF,)),
        ],
        compiler_params=pltpu.CompilerParams(use_tc_tiling_on_sc=False),
    )
    def gather_kernel(idx_hbm, table_hbm, out_hbm, idx_all, rows, gsems, ssems):
        wid = lax.axis_index("s") * NC + lax.axis_index("c")
        base = wid * per_w

        pltpu.sync_copy(idx_hbm.at[pl.ds(wid * n_chunks, n_chunks)], idx_all)

        def fire(b, s):
            for k in range(_SUP):
                pltpu.async_copy(
                    table_hbm.at[idx_all.at[s * _SUP + k]],
                    rows.at[b, pl.ds(k * _C, _C)],
                    gsems.at[b],
                )

        def wait_gather(b):
            pltpu.make_async_copy(
                table_hbm.at[pl.ds(0, CH)], rows.at[b], gsems.at[b]
            ).wait()

        def store_start(b, s):
            pltpu.async_copy(
                rows.at[b], out_hbm.at[pl.ds(base + s * CH, CH)], ssems.at[b]
            )

        def wait_store(b):
            pltpu.make_async_copy(
                rows.at[b], out_hbm.at[pl.ds(0, CH)], ssems.at[b]
            ).wait()

        fire(0, 0)

        def body(i, carry):
            for b in range(_NBUF):
                s = i * _NBUF + b
                nxt = s + 1
                bf = (b + 1) % _NBUF

                @pl.when(s >= 1)
                def _():
                    wait_store(bf)

                @pl.when(nxt < n_super)
                def _():
                    fire(bf, nxt)

                wait_gather(b)
                store_start(b, s)
            return carry

        lax.fori_loop(0, n_super // _NBUF, body, 0)
        wait_store((n_super - 1) % _NBUF)

    return gather_kernel


def kernel(x, weight):
    B = x.shape[0] * x.shape[1]
    V, D = weight.shape
    idx2d = x.reshape(B // _C, _C)
    out = _build(B, V, D)(idx2d, weight)
    return out.reshape(x.shape + (D,))
